# baseline (speedup 1.0000x reference)
import numpy as np
import jax
import jax.numpy as jnp
from jax.experimental import pallas as pl
from jax.experimental.pallas import tpu as pltpu

_ALPHA = 4
_N_FAST_CHUNKS = 4


def _dma_body(frames_ref, slow_ref, fast_ref, sems):
    T = frames_ref.shape[1]
    n_slow = T // _ALPHA
    span = T // _N_FAST_CHUNKS
    copies = []
    for g in range(_N_FAST_CHUNKS):
        copies.append(
            pltpu.make_async_copy(
                frames_ref.at[:, pl.ds(g * span, span)],
                fast_ref.at[:, pl.ds(g * span, span)],
                sems.at[g],
            )
        )
    for s in range(n_slow):
        i = (s * (T - 1)) // (n_slow - 1)
        copies.append(
            pltpu.make_async_copy(
                frames_ref.at[:, pl.ds(i, 1)],
                slow_ref.at[:, pl.ds(s, 1)],
                sems.at[_N_FAST_CHUNKS + s],
            )
        )
    for c in copies:
        c.start()
    for c in copies:
        c.wait()


def kernel(frames):
    C, T, H, W = frames.shape
    n_slow = T // _ALPHA
    idx = np.linspace(0.0, T - 1, n_slow).astype(np.int32)
    assert all(int(i) == (s * (T - 1)) // (n_slow - 1) for s, i in enumerate(idx))

    slow, fast = pl.pallas_call(
        _dma_body,
        in_specs=[pl.BlockSpec(memory_space=pl.ANY)],
        out_specs=[
            pl.BlockSpec(memory_space=pl.ANY),
            pl.BlockSpec(memory_space=pl.ANY),
        ],
        out_shape=[
            jax.ShapeDtypeStruct((C, n_slow, H, W), frames.dtype),
            jax.ShapeDtypeStruct((C, T, H, W), frames.dtype),
        ],
        scratch_shapes=[pltpu.SemaphoreType.DMA((_N_FAST_CHUNKS + n_slow,))],
    )(frames)
    return (slow, fast)


# speedup vs baseline: 52.8080x; 52.8080x over previous
import numpy as np
import jax
import jax.numpy as jnp
from jax.experimental import pallas as pl
from jax.experimental.pallas import tpu as pltpu

_ALPHA = 4
_GROUP = 8
_SLOW_PER_GROUP = _GROUP // _ALPHA
_HSPLIT = 2


def _pack_body(frames_ref, slow_ref, fast_ref):
    g = pl.program_id(0)
    n_slow = pl.num_programs(0) * _SLOW_PER_GROUP
    T = n_slow * _ALPHA
    fast_ref[...] = frames_ref[...]
    for u in range(_SLOW_PER_GROUP):
        s = g * _SLOW_PER_GROUP + u
        off = (s * (T - 1)) // (n_slow - 1) - _GROUP * g
        slow_ref[:, pl.ds(u, 1), :, :] = frames_ref[:, pl.ds(off, 1), :, :]


def kernel(frames):
    C, T, H, W = frames.shape
    n_slow = T // _ALPHA
    n_groups = T // _GROUP
    idx = np.linspace(0.0, T - 1, n_slow).astype(np.int32)
    assert all(int(i) == (s * (T - 1)) // (n_slow - 1) for s, i in enumerate(idx))
    assert all(_GROUP * (s // _SLOW_PER_GROUP) <= int(i) < _GROUP * (s // _SLOW_PER_GROUP + 1)
               for s, i in enumerate(idx))
    hb = H // _HSPLIT

    def group_map(g, h):
        return (0, g, h, 0)

    slow, fast = pl.pallas_call(
        _pack_body,
        grid=(n_groups, _HSPLIT),
        in_specs=[pl.BlockSpec((C, _GROUP, hb, W), group_map)],
        out_specs=[
            pl.BlockSpec((C, _SLOW_PER_GROUP, hb, W), group_map),
            pl.BlockSpec((C, _GROUP, hb, W), group_map),
        ],
        out_shape=[
            jax.ShapeDtypeStruct((C, n_slow, H, W), frames.dtype),
            jax.ShapeDtypeStruct((C, T, H, W), frames.dtype),
        ],
        compiler_params=pltpu.CompilerParams(vmem_limit_bytes=64 * 1024 * 1024),
    )(frames)
    return (slow, fast)
